# baseline (speedup 1.0000x reference)
import functools

import jax
import jax.numpy as jnp
from jax import lax
from jax.experimental import pallas as pl
from jax.experimental.pallas import tpu as pltpu
from jax.experimental.pallas import tpu_sc as plsc

K = 16
QT = 256
ROWW = 128
OUTW = 3 + 64


def _topk_body(xyz_ref, sxyz_ref, idx_ref):
    q = xyz_ref[0]
    s = sxyz_ref[0]
    n = s.shape[0]
    d = -2.0 * lax.dot_general(s, q, (((1,), (1,)), ((), ())),
                               preferred_element_type=jnp.float32)
    q2 = jnp.sum(q * q, axis=1)
    s2 = jnp.sum(s * s, axis=1)
    d = d + q2[None, :]
    d = d + s2[:, None]
    iota = lax.broadcasted_iota(jnp.int32, d.shape, 0)
    inf = jnp.float32(jnp.inf)
    for k in range(K):
        w = jnp.min(d, axis=0)
        wi = jnp.min(jnp.where(d == w[None, :], iota, n), axis=0)
        idx_ref[0, k, :] = wi
        d = jnp.where(iota == wi[None, :], inf, d)


def _topk(s_xyz, xyz):
    B, N, _ = s_xyz.shape
    S = xyz.shape[1]
    return pl.pallas_call(
        _topk_body,
        grid=(B, S // QT),
        in_specs=[
            pl.BlockSpec((1, QT, 3), lambda b, i: (b, i, 0)),
            pl.BlockSpec((1, N, 3), lambda b, i: (b, 0, 0)),
        ],
        out_specs=pl.BlockSpec((1, K, QT), lambda b, i: (b, 0, i)),
        out_shape=jax.ShapeDtypeStruct((B, K, S), jnp.int32),
    )(xyz, s_xyz)


def _make_sc_gather(BS):
    NC, NS = 2, 16
    NW = NC * NS
    QW = BS // NW
    NQ = 8
    NB = QW // NQ
    mesh = plsc.VectorSubcoreMesh(core_axis_name="c", subcore_axis_name="s")

    @functools.partial(
        pl.kernel, mesh=mesh,
        out_type=[
            jax.ShapeDtypeStruct((BS * K * OUTW,), jnp.float32),
            jax.ShapeDtypeStruct((BS * K * 3,), jnp.float32),
        ],
        scratch_types=[
            pltpu.VMEM((NQ * K,), jnp.int32),
            pltpu.VMEM((NQ * K, ROWW), jnp.float32),
            pltpu.VMEM((NQ, 16), jnp.float32),
            pltpu.VMEM((NQ * K * OUTW + 16,), jnp.float32),
            pltpu.VMEM((NQ * K * 3 + 16,), jnp.float32),
            pltpu.SemaphoreType.DMA,
        ],
    )
    def sc_gather(feat_hbm, gidx_hbm, qpad_hbm, newp_hbm, gxyz_hbm,
                  idx_v, rows_v, q_v, out_v, gx_v, sem):
        wid = lax.axis_index("s") * NC + lax.axis_index("c")

        def block(t, _):
            qbase = wid * QW + t * NQ
            pltpu.sync_copy(gidx_hbm.at[pl.ds(qbase * K, NQ * K)], idx_v)
            pltpu.async_copy(feat_hbm.at[idx_v], rows_v, sem).wait()
            pltpu.sync_copy(qpad_hbm.at[pl.ds(qbase, NQ)], q_v)

            def body(i, _):
                qvec = q_v[i, :]
                for r in range(K):
                    row = i * K + r
                    d0 = row * OUTW
                    v0 = rows_v[row, pl.ds(0, 16)] - qvec
                    out_v[pl.ds(d0, 16)] = v0
                    for j in range(1, 5):
                        out_v[pl.ds(d0 + 16 * j, 16)] = rows_v[row, pl.ds(16 * j, 16)]
                    gx_v[pl.ds(row * 3, 16)] = v0
                return 0

            lax.fori_loop(0, NQ, body, 0)
            pltpu.sync_copy(out_v.at[pl.ds(0, NQ * K * OUTW)],
                            newp_hbm.at[pl.ds(qbase * K * OUTW, NQ * K * OUTW)])
            pltpu.sync_copy(gx_v.at[pl.ds(0, NQ * K * 3)],
                            gxyz_hbm.at[pl.ds(qbase * K * 3, NQ * K * 3)])
            return 0

        lax.fori_loop(0, NB, block, 0)

    return sc_gather


def kernel(s_xyz, xyz, s_points, nsample):
    B, N, _ = s_xyz.shape
    S = xyz.shape[1]
    D = s_points.shape[2]
    BS = B * S

    idx = _topk(s_xyz, xyz)
    idx = jnp.transpose(idx, (0, 2, 1))

    pad = jnp.zeros((B, N, ROWW - 3 - D), jnp.float32)
    feat = jnp.concatenate([s_xyz, s_points, pad], axis=-1).reshape(B * N, ROWW)
    gidx = (idx + (jnp.arange(B, dtype=jnp.int32) * N)[:, None, None]
            ).reshape(BS * K)
    qpad = jnp.concatenate(
        [xyz, jnp.zeros((B, S, 13), jnp.float32)], axis=-1).reshape(BS, 16)

    newp_flat, gxyz_flat = _make_sc_gather(BS)(feat, gidx, qpad)
    new_points = newp_flat.reshape(B, S, K, OUTW)
    grouped_xyz_norm = gxyz_flat.reshape(B, S, K, 3)
    return new_points, grouped_xyz_norm


# speedup vs baseline: 2.1496x; 2.1496x over previous
import functools

import jax
import jax.numpy as jnp
from jax import lax
from jax.experimental import pallas as pl
from jax.experimental.pallas import tpu as pltpu
from jax.experimental.pallas import tpu_sc as plsc

K = 16
QT = 256
ROWW = 128
OUTW = 3 + 64


G = 128


def _topk_body(xyz_ref, sxyz_ref, idx_ref):
    q = xyz_ref[0]
    s = sxyz_ref[0]
    n = s.shape[0]
    gs = n // G
    d = -2.0 * lax.dot_general(s, q, (((1,), (1,)), ((), ())),
                               preferred_element_type=jnp.float32)
    q2 = jnp.sum(q * q, axis=1)
    s2 = jnp.sum(s * s, axis=1)
    d = d + q2[None, :]
    d = d + s2[:, None]
    qt = d.shape[1]
    d3 = d.reshape(G, gs, qt)
    gidx = (lax.broadcasted_iota(jnp.int32, (G, gs, qt), 0) * gs
            + lax.broadcasted_iota(jnp.int32, (G, gs, qt), 1))
    inf = jnp.float32(jnp.inf)

    def round_(d3):
        gmin = jnp.min(d3, axis=1)
        gam = jnp.min(jnp.where(d3 == gmin[:, None, :], gidx, n), axis=1)
        d3 = jnp.where(gidx == gam[:, None, :], inf, d3)
        return d3, gmin, gam

    def merge(W, WI, cv, ci):
        ev = jnp.concatenate([W, cv], axis=0)
        ei = jnp.concatenate([WI, ci], axis=0)
        nW, nWI = [], []
        for _ in range(K):
            w = jnp.min(ev, axis=0)
            wm = ev == w[None, :]
            wi = jnp.min(jnp.where(wm, ei, n), axis=0)
            nW.append(w)
            nWI.append(wi)
            ev = jnp.where(wm & (ei == wi[None, :]), inf, ev)
        return jnp.stack(nW), jnp.stack(nWI)

    d3, cv, ci = round_(d3)
    W, WI = merge(jnp.full((K, qt), inf), jnp.full((K, qt), n, jnp.int32),
                  cv, ci)
    for _ in range(2):
        d3, cv, ci = round_(d3)
        W, WI = merge(W, WI, cv, ci)

    def cond(st):
        r, done = st[0], st[1]
        return jnp.logical_and(r < K, jnp.logical_not(done))

    def body(st):
        r, _, d3, W, WI = st
        d3, cv, ci = round_(d3)
        W, WI = merge(W, WI, cv, ci)
        t, ti = W[K - 1], WI[K - 1]
        bad = (d3 < t[None, None, :]) | ((d3 == t[None, None, :])
                                         & (gidx < ti[None, None, :]))
        return r + 1, jnp.logical_not(jnp.any(bad)), d3, W, WI

    t, ti = W[K - 1], WI[K - 1]
    bad = (d3 < t[None, None, :]) | ((d3 == t[None, None, :])
                                     & (gidx < ti[None, None, :]))
    st = (jnp.int32(3), jnp.logical_not(jnp.any(bad)), d3, W, WI)
    _, _, _, W, WI = lax.while_loop(cond, body, st)
    idx_ref[0] = WI


def _topk(s_xyz, xyz):
    B, N, _ = s_xyz.shape
    S = xyz.shape[1]
    return pl.pallas_call(
        _topk_body,
        grid=(B, S // QT),
        in_specs=[
            pl.BlockSpec((1, QT, 3), lambda b, i: (b, i, 0)),
            pl.BlockSpec((1, N, 3), lambda b, i: (b, 0, 0)),
        ],
        out_specs=pl.BlockSpec((1, K, QT), lambda b, i: (b, 0, i)),
        out_shape=jax.ShapeDtypeStruct((B, K, S), jnp.int32),
    )(xyz, s_xyz)


def _make_sc_gather(BS):
    NC, NS = 2, 16
    NW = NC * NS
    QW = BS // NW
    NQ = 8
    NB = QW // NQ
    mesh = plsc.VectorSubcoreMesh(core_axis_name="c", subcore_axis_name="s")

    @functools.partial(
        pl.kernel, mesh=mesh,
        out_type=[
            jax.ShapeDtypeStruct((BS * K * OUTW,), jnp.float32),
            jax.ShapeDtypeStruct((BS * K * 3,), jnp.float32),
        ],
        scratch_types=[
            pltpu.VMEM((NQ * K,), jnp.int32),
            pltpu.VMEM((NQ * K, ROWW), jnp.float32),
            pltpu.VMEM((NQ, 16), jnp.float32),
            pltpu.VMEM((NQ * K * OUTW + 16,), jnp.float32),
            pltpu.VMEM((NQ * K * 3 + 16,), jnp.float32),
            pltpu.SemaphoreType.DMA,
        ],
    )
    def sc_gather(feat_hbm, gidx_hbm, qpad_hbm, newp_hbm, gxyz_hbm,
                  idx_v, rows_v, q_v, out_v, gx_v, sem):
        wid = lax.axis_index("s") * NC + lax.axis_index("c")

        def block(t, _):
            qbase = wid * QW + t * NQ
            pltpu.sync_copy(gidx_hbm.at[pl.ds(qbase * K, NQ * K)], idx_v)
            pltpu.async_copy(feat_hbm.at[idx_v], rows_v, sem).wait()
            pltpu.sync_copy(qpad_hbm.at[pl.ds(qbase, NQ)], q_v)

            def body(i, _):
                qvec = q_v[i, :]
                for r in range(K):
                    row = i * K + r
                    d0 = row * OUTW
                    v0 = rows_v[row, pl.ds(0, 16)] - qvec
                    out_v[pl.ds(d0, 16)] = v0
                    for j in range(1, 5):
                        out_v[pl.ds(d0 + 16 * j, 16)] = rows_v[row, pl.ds(16 * j, 16)]
                    gx_v[pl.ds(row * 3, 16)] = v0
                return 0

            lax.fori_loop(0, NQ, body, 0)
            pltpu.sync_copy(out_v.at[pl.ds(0, NQ * K * OUTW)],
                            newp_hbm.at[pl.ds(qbase * K * OUTW, NQ * K * OUTW)])
            pltpu.sync_copy(gx_v.at[pl.ds(0, NQ * K * 3)],
                            gxyz_hbm.at[pl.ds(qbase * K * 3, NQ * K * 3)])
            return 0

        lax.fori_loop(0, NB, block, 0)

    return sc_gather


def kernel(s_xyz, xyz, s_points, nsample):
    B, N, _ = s_xyz.shape
    S = xyz.shape[1]
    D = s_points.shape[2]
    BS = B * S

    idx = _topk(s_xyz, xyz)
    idx = jnp.transpose(idx, (0, 2, 1))

    pad = jnp.zeros((B, N, ROWW - 3 - D), jnp.float32)
    feat = jnp.concatenate([s_xyz, s_points, pad], axis=-1).reshape(B * N, ROWW)
    gidx = (idx + (jnp.arange(B, dtype=jnp.int32) * N)[:, None, None]
            ).reshape(BS * K)
    qpad = jnp.concatenate(
        [xyz, jnp.zeros((B, S, 13), jnp.float32)], axis=-1).reshape(BS, 16)

    newp_flat, gxyz_flat = _make_sc_gather(BS)(feat, gidx, qpad)
    new_points = newp_flat.reshape(B, S, K, OUTW)
    grouped_xyz_norm = gxyz_flat.reshape(B, S, K, 3)
    return new_points, grouped_xyz_norm
